# baseline (speedup 1.0000x reference)
import functools

import jax
import jax.numpy as jnp
from jax import lax
from jax.experimental import pallas as pl
from jax.experimental.pallas import tpu as pltpu
from jax.experimental.pallas import tpu_sc as plsc

SEQ = 200
BATCH = 4096
DIM = 64
B = SEQ * BATCH

NC = 2
NS = 16
NW = NC * NS
BPW = B // NW

KROWS = 128
NK = BPW // KROWS
CH = 4
CROWS = KROWS * CH
NCHUNK = NK // CH

_MESH = plsc.VectorSubcoreMesh(
    core_axis_name="c", subcore_axis_name="s", num_cores=NC, num_subcores=NS
)


@functools.partial(
    pl.kernel,
    out_type=jax.ShapeDtypeStruct((B, DIM), jnp.float32),
    mesh=_MESH,
    compiler_params=pltpu.CompilerParams(use_tc_tiling_on_sc=False),
    scratch_types=[
        pltpu.VMEM((NK, KROWS), jnp.int32),
        pltpu.VMEM((CROWS, DIM), jnp.float32),
        pltpu.VMEM((CROWS, DIM), jnp.float32),
        pltpu.SemaphoreType.DMA,
        pltpu.SemaphoreType.DMA,
    ],
)
def _gather_kernel(table_hbm, idx_hbm, out_hbm, idx_v, buf0, buf1, sem0, sem1):
    wid = lax.axis_index("s") * NC + lax.axis_index("c")
    pltpu.sync_copy(idx_hbm.at[wid], idx_v)
    out_base = wid * BPW

    @pl.loop(0, NCHUNK, step=2)
    def _chunks(g):
        copies = []
        for p, (buf, sem) in enumerate(((buf0, sem0), (buf1, sem1))):
            per_stream = []
            for j in range(CH):
                k = (g + p) * CH + j
                per_stream.append(
                    pltpu.async_copy(
                        table_hbm.at[idx_v.at[k]],
                        buf.at[pl.ds(j * KROWS, KROWS)],
                        sem,
                    )
                )
            copies.append(per_stream)
        for p, (buf, _) in enumerate(((buf0, sem0), (buf1, sem1))):
            for c in copies[p]:
                c.wait()
            pltpu.sync_copy(
                buf, out_hbm.at[pl.ds(out_base + (g + p) * CROWS, CROWS)]
            )


def kernel(input, table):
    idx = input.reshape(NW, NK, KROWS)
    out = _gather_kernel(table, idx)
    return out.reshape(SEQ, BATCH, DIM)
